# Initial kernel scaffold; baseline (speedup 1.0000x reference)
import jax
import jax.numpy as jnp
from jax.experimental import pallas as pl


def kernel(x, weight, bias, style_ids):
    raise NotImplementedError("write your pallas kernel here")


# speedup vs baseline: 1.8904x; 1.8904x over previous
import jax
import jax.numpy as jnp
from jax.experimental import pallas as pl
from jax.experimental.pallas import tpu as pltpu

EPS = 1e-05


def _cin_kernel(sid_ref, x_ref, w_ref, b_ref, o_ref):
    b = pl.program_id(0)
    sid = sid_ref[b]
    x = x_ref[0]
    mean = jnp.mean(x, axis=-1, keepdims=True)
    xc = x - mean
    var = jnp.mean(xc * xc, axis=-1, keepdims=True)
    inv = jax.lax.rsqrt(var + EPS)
    s = w_ref.shape[1]
    sel = jax.lax.broadcasted_iota(jnp.int32, (1, s), 1) == sid
    w = jnp.sum(jnp.where(sel, w_ref[...], 0.0), axis=1, keepdims=True)
    bb = jnp.sum(jnp.where(sel, b_ref[...], 0.0), axis=1, keepdims=True)
    o_ref[0] = xc * (inv * w) + bb


def kernel(x, weight, bias, style_ids):
    B, C, L = x.shape
    S = weight.shape[0]
    wt = weight.T
    bt = bias.T
    sids = style_ids.astype(jnp.int32)
    return pl.pallas_call(
        _cin_kernel,
        out_shape=jax.ShapeDtypeStruct((B, C, L), x.dtype),
        grid_spec=pltpu.PrefetchScalarGridSpec(
            num_scalar_prefetch=1,
            grid=(B,),
            in_specs=[
                pl.BlockSpec((1, C, L), lambda b, sids_ref: (b, 0, 0)),
                pl.BlockSpec((C, S), lambda b, sids_ref: (0, 0)),
                pl.BlockSpec((C, S), lambda b, sids_ref: (0, 0)),
            ],
            out_specs=pl.BlockSpec((1, C, L), lambda b, sids_ref: (b, 0, 0)),
        ),
        compiler_params=pltpu.CompilerParams(
            dimension_semantics=("parallel",),
        ),
        name="conditional_instance_norm1d",
    )(sids, x, wt, bt)
